# Initial kernel scaffold; baseline (speedup 1.0000x reference)
import jax
import jax.numpy as jnp
from jax.experimental import pallas as pl


def kernel(hidden_states, top_k_index, top_k_weights, gate_up_proj, down_proj):
    raise NotImplementedError("write your pallas kernel here")


# speedup vs baseline: 1.6731x; 1.6731x over previous
import functools

import jax
import jax.numpy as jnp
from jax import lax
from jax.experimental import pallas as pl
from jax.experimental.pallas import tpu as pltpu
from jax.experimental.pallas import tpu_sc as plsc

E = 8
H = 2048
I = 1408
T = 8192
K = 2
P = T * K

BM = 256
TMAX = P // BM + E
GW = 16


def _sc_gather_rows(table, idx, n_rows, row_dim):
    mesh = plsc.VectorSubcoreMesh(core_axis_name="core", subcore_axis_name="subcore")
    info = plsc.get_sparse_core_info()
    nw = info.num_cores * info.num_subcores
    rows_w = n_rows // nw
    nchunks = rows_w // GW

    @functools.partial(
        pl.kernel,
        out_type=jax.ShapeDtypeStruct((n_rows, row_dim), jnp.float32),
        mesh=mesh,
        scratch_types=[
            pltpu.VMEM((rows_w,), jnp.int32),
            pltpu.VMEM((2, GW, row_dim), jnp.float32),
            pltpu.SemaphoreType.DMA((2,)),
            pltpu.SemaphoreType.DMA((2,)),
        ],
    )
    def gather_kernel(tbl_hbm, i_hbm, o_hbm, idx_v, bufs, gsem, wsem):
        wid = lax.axis_index("subcore") * info.num_cores + lax.axis_index("core")
        base = wid * rows_w
        pltpu.sync_copy(i_hbm.at[pl.ds(base, rows_w)], idx_v)

        def start_gather(c, b):
            pltpu.async_copy(
                tbl_hbm.at[idx_v.at[pl.ds(c * GW, GW)]], bufs.at[b], gsem.at[b])

        def wait_gather(b):
            pltpu.make_async_copy(
                tbl_hbm.at[pl.ds(0, GW)], bufs.at[b], gsem.at[b]).wait()

        def start_write(c, b):
            pltpu.async_copy(
                bufs.at[b], o_hbm.at[pl.ds(base + c * GW, GW)], wsem.at[b])

        def wait_write(b):
            pltpu.make_async_copy(
                bufs.at[b], o_hbm.at[pl.ds(base, GW)], wsem.at[b]).wait()

        start_gather(0, 0)
        start_gather(1, 1)

        @pl.loop(0, nchunks, step=2)
        def _(c):
            for b in range(2):
                wait_gather(b)
                start_write(c + b, b)
            for b in range(2):
                wait_write(b)

                @pl.when(c + 2 + b < nchunks)
                def _():
                    start_gather(c + 2 + b, b)

    return gather_kernel(table, idx)


def _mlp_tile_body(meta_ref, xs_ref, gw_ref, uw_ref, dw_ref, o_ref):
    i = pl.program_id(0)
    blk = meta_ref[1, i]
    row_s = meta_ref[2, i]
    row_e = meta_ref[3, i]
    init = meta_ref[4, i]

    x = xs_ref[...].astype(jnp.bfloat16)
    gw = gw_ref[0, 0]
    uw = uw_ref[0, 0]
    dw = dw_ref[0]

    gate = jnp.dot(x, gw, preferred_element_type=jnp.float32)
    up = jnp.dot(x, uw, preferred_element_type=jnp.float32)
    h = (gate * jax.nn.sigmoid(gate)) * up
    y = jnp.dot(h.astype(jnp.bfloat16), dw, preferred_element_type=jnp.float32)

    rows = blk * BM + lax.broadcasted_iota(jnp.int32, (BM, 1), 0)
    mask = (rows >= row_s) & (rows < row_e)
    y = jnp.where(mask, y, 0.0)

    @pl.when(init == 1)
    def _():
        o_ref[...] = y

    @pl.when(init == 0)
    def _():
        o_ref[...] += y


def _grouped_mlp(xs, gup_t, down_t, meta):
    grid_spec = pltpu.PrefetchScalarGridSpec(
        num_scalar_prefetch=1,
        grid=(TMAX,),
        in_specs=[
            pl.BlockSpec((BM, H), lambda i, m: (m[1, i], 0)),
            pl.BlockSpec((1, 1, H, I), lambda i, m: (m[0, i], 0, 0, 0)),
            pl.BlockSpec((1, 1, H, I), lambda i, m: (m[0, i], 1, 0, 0)),
            pl.BlockSpec((1, I, H), lambda i, m: (m[0, i], 0, 0)),
        ],
        out_specs=pl.BlockSpec((BM, H), lambda i, m: (m[1, i], 0)),
    )
    return pl.pallas_call(
        _mlp_tile_body,
        grid_spec=grid_spec,
        out_shape=jax.ShapeDtypeStruct((P, H), jnp.float32),
    )(meta, xs, gup_t, gup_t, down_t)


def _combine_body(g_ref, w_ref, o_ref):
    g = g_ref[...]
    w = w_ref[...]
    o_ref[...] = g[:, 0, :] * w[0][:, None] + g[:, 1, :] * w[1][:, None]


def _combine(g, w_t):
    BC = 512
    return pl.pallas_call(
        _combine_body,
        grid=(T // BC,),
        in_specs=[
            pl.BlockSpec((BC, 2, H), lambda i: (i, 0, 0)),
            pl.BlockSpec((2, BC), lambda i: (0, i)),
        ],
        out_specs=pl.BlockSpec((BC, H), lambda i: (i, 0)),
        out_shape=jax.ShapeDtypeStruct((T, H), jnp.float32),
    )(g, w_t)


def _routing_metadata(top_k_index):
    flat_e = top_k_index.reshape(-1).astype(jnp.int32)
    sort_idx = jnp.argsort(flat_e, stable=True)
    sorted_token = (sort_idx // K).astype(jnp.int32)
    inv = jnp.argsort(sort_idx).astype(jnp.int32)

    sizes = jnp.sum(flat_e[:, None] == jnp.arange(E, dtype=jnp.int32)[None, :],
                    axis=0).astype(jnp.int32)
    ends = jnp.cumsum(sizes)
    starts = ends - sizes

    first_blk = starts // BM
    last_blk = jnp.maximum(ends - 1, 0) // BM
    ntiles = jnp.where(sizes > 0, last_blk - first_blk + 1, 0)
    tile_end = jnp.cumsum(ntiles)
    tile_start = tile_end - ntiles

    tile_ids = jnp.arange(TMAX, dtype=jnp.int32)
    grp = jnp.sum(tile_ids[:, None] >= tile_end[None, :], axis=1).astype(jnp.int32)
    valid = grp < E
    grp_c = jnp.minimum(grp, E - 1)
    blk = jnp.where(valid, first_blk[grp_c] + (tile_ids - tile_start[grp_c]),
                    P // BM - 1)
    row_s = jnp.where(valid, jnp.maximum(starts[grp_c], blk * BM), 0)
    row_e = jnp.where(valid, jnp.minimum(ends[grp_c], blk * BM + BM), 0)
    init = jnp.concatenate([jnp.ones((1,), jnp.int32),
                            (blk[1:] != blk[:-1]).astype(jnp.int32)])
    meta = jnp.stack([grp_c.astype(jnp.int32), blk.astype(jnp.int32),
                      row_s.astype(jnp.int32), row_e.astype(jnp.int32), init])
    return sort_idx, sorted_token, inv, meta


@jax.jit
def kernel(hidden_states, top_k_index, top_k_weights, gate_up_proj, down_proj):
    _, sorted_token, inv, meta = _routing_metadata(top_k_index)

    gup_t = gate_up_proj.reshape(E, 2, I, H).transpose(0, 1, 3, 2).astype(jnp.bfloat16)
    down_t = down_proj.transpose(0, 2, 1).astype(jnp.bfloat16)

    xs = _sc_gather_rows(hidden_states, sorted_token, P, H)
    ys = _grouped_mlp(xs, gup_t, down_t, meta)
    g = _sc_gather_rows(ys, inv, P, H).reshape(T, K, H)
    w_t = top_k_weights.astype(jnp.float32).T
    return _combine(g, w_t)


# speedup vs baseline: 1.8419x; 1.1009x over previous
import functools

import jax
import jax.numpy as jnp
from jax import lax
from jax.experimental import pallas as pl
from jax.experimental.pallas import tpu as pltpu
from jax.experimental.pallas import tpu_sc as plsc

E = 8
H = 2048
I = 1408
T = 8192
K = 2
P = T * K

BM = 256
PC = P // 2
NBC = PC // BM
TMAXC = NBC + E
T2 = T // 2
BC = 512
GW = 16


def _sc_gather_rows(table, idx, n_rows, row_dim):
    mesh = plsc.VectorSubcoreMesh(core_axis_name="core", subcore_axis_name="subcore")
    info = plsc.get_sparse_core_info()
    nw = info.num_cores * info.num_subcores
    rows_w = n_rows // nw
    nchunks = rows_w // GW

    @functools.partial(
        pl.kernel,
        out_type=jax.ShapeDtypeStruct((n_rows, row_dim), table.dtype),
        mesh=mesh,
        scratch_types=[
            pltpu.VMEM((rows_w,), jnp.int32),
            pltpu.VMEM((2, GW, row_dim), table.dtype),
            pltpu.SemaphoreType.DMA((2,)),
            pltpu.SemaphoreType.DMA((2,)),
        ],
    )
    def gather_kernel(tbl_hbm, i_hbm, o_hbm, idx_v, bufs, gsem, wsem):
        wid = lax.axis_index("subcore") * info.num_cores + lax.axis_index("core")
        base = wid * rows_w
        pltpu.sync_copy(i_hbm.at[pl.ds(base, rows_w)], idx_v)

        def start_gather(c, b):
            pltpu.async_copy(
                tbl_hbm.at[idx_v.at[pl.ds(c * GW, GW)]], bufs.at[b], gsem.at[b])

        def wait_gather(b):
            pltpu.make_async_copy(
                tbl_hbm.at[pl.ds(0, GW)], bufs.at[b], gsem.at[b]).wait()

        def start_write(c, b):
            pltpu.async_copy(
                bufs.at[b], o_hbm.at[pl.ds(base + c * GW, GW)], wsem.at[b])

        def wait_write(b):
            pltpu.make_async_copy(
                bufs.at[b], o_hbm.at[pl.ds(base, GW)], wsem.at[b]).wait()

        start_gather(0, 0)
        start_gather(1, 1)

        @pl.loop(0, nchunks, step=2)
        def _(c):
            for b in range(2):
                wait_gather(b)
                start_write(c + b, b)
            for b in range(2):
                wait_write(b)

                @pl.when(c + 2 + b < nchunks)
                def _():
                    start_gather(c + 2 + b, b)

    return gather_kernel(table, idx)


def _mlp_tile_body(*refs):
    meta_ref, xs_ref, gw_ref, uw_ref, dw_ref = refs[:5]
    o_ref = refs[-1]
    i = pl.program_id(0)
    blk = meta_ref[1, i]
    row_s = meta_ref[2, i]
    row_e = meta_ref[3, i]
    init = meta_ref[4, i]

    @pl.when(row_s < row_e)
    def _():
        x = xs_ref[...].astype(jnp.bfloat16)
        gw = gw_ref[0]
        uw = uw_ref[0]
        dw = dw_ref[0]

        nt = (((1,), (1,)), ((), ()))
        gate = lax.dot_general(x, gw, nt, preferred_element_type=jnp.float32)
        up = lax.dot_general(x, uw, nt, preferred_element_type=jnp.float32)
        h = (gate * jax.nn.sigmoid(gate)) * up
        y = lax.dot_general(h.astype(jnp.bfloat16), dw, nt,
                            preferred_element_type=jnp.float32)

        rows = blk * BM + lax.broadcasted_iota(jnp.int32, (BM, 1), 0)
        mask = (rows >= row_s) & (rows < row_e)
        y = jnp.where(mask, y, 0.0)

        @pl.when(init == 1)
        def _():
            o_ref[...] = y

        @pl.when(init == 0)
        def _():
            o_ref[...] += y


def _grouped_mlp_chunk(xs_h, gup_b, down_b, meta, out_blk_off, prev=None):
    in_specs = [
        pl.BlockSpec((BM, H), lambda i, m: (m[1, i], 0)),
        pl.BlockSpec((1, I, H), lambda i, m: (m[0, i], 0, 0)),
        pl.BlockSpec((1, I, H), lambda i, m: (m[0, i], 1, 0)),
        pl.BlockSpec((1, H, I), lambda i, m: (m[0, i], 0, 0)),
    ]
    args = [meta, xs_h, gup_b, gup_b, down_b]
    aliases = {}
    if prev is not None:
        in_specs.append(pl.BlockSpec(memory_space=pl.ANY))
        args.append(prev)
        aliases = {5: 0}
    grid_spec = pltpu.PrefetchScalarGridSpec(
        num_scalar_prefetch=1,
        grid=(TMAXC,),
        in_specs=in_specs,
        out_specs=pl.BlockSpec(
            (BM, H), lambda i, m, off=out_blk_off: (m[1, i] + off, 0)),
    )
    return pl.pallas_call(
        _mlp_tile_body,
        grid_spec=grid_spec,
        out_shape=jax.ShapeDtypeStruct((P, H), jnp.float32),
        input_output_aliases=aliases,
    )(*args)


def _combine_body(*refs):
    g0_ref, g1_ref, w_ref = refs[:3]
    o_ref = refs[-1]
    w = w_ref[...]
    o_ref[...] = g0_ref[0] * w[0][:, None] + g1_ref[0] * w[1][:, None]


def _combine_half(g, w_h, out_blk_off, prev=None):
    in_specs = [
        pl.BlockSpec((1, BC, H), lambda i: (0, i, 0)),
        pl.BlockSpec((1, BC, H), lambda i: (1, i, 0)),
        pl.BlockSpec((2, BC), lambda i: (0, i)),
    ]
    args = [g, g, w_h]
    aliases = {}
    if prev is not None:
        in_specs.append(pl.BlockSpec(memory_space=pl.ANY))
        args.append(prev)
        aliases = {3: 0}
    return pl.pallas_call(
        _combine_body,
        grid=(T2 // BC,),
        in_specs=in_specs,
        out_specs=pl.BlockSpec(
            (BC, H), lambda i, off=out_blk_off: (i + off, 0)),
        out_shape=jax.ShapeDtypeStruct((T, H), jnp.float32),
        input_output_aliases=aliases,
    )(*args)


def _tile_meta(starts, ends, lo, hi):
    s = jnp.clip(starts, lo, hi) - lo
    e = jnp.clip(ends, lo, hi) - lo
    sizes = e - s
    nb = (hi - lo) // BM
    first_blk = s // BM
    last_blk = jnp.maximum(e - 1, 0) // BM
    ntiles = jnp.where(sizes > 0, last_blk - first_blk + 1, 0)
    tile_end = jnp.cumsum(ntiles)
    tile_start = tile_end - ntiles

    ids = jnp.arange(TMAXC, dtype=jnp.int32)
    grp = jnp.sum(ids[:, None] >= tile_end[None, :], axis=1).astype(jnp.int32)
    valid = grp < E
    g = jnp.minimum(grp, E - 1)
    blk = jnp.where(valid, first_blk[g] + (ids - tile_start[g]), nb - 1)
    row_s = jnp.where(valid, jnp.maximum(s[g], blk * BM), 0)
    row_e = jnp.where(valid, jnp.minimum(e[g], blk * BM + BM), 0)
    init = jnp.concatenate([jnp.ones((1,), jnp.int32),
                            (blk[1:] != blk[:-1]).astype(jnp.int32)])
    return jnp.stack([g.astype(jnp.int32), blk.astype(jnp.int32),
                      row_s.astype(jnp.int32), row_e.astype(jnp.int32), init])


def _routing_metadata(top_k_index):
    flat_e = top_k_index.reshape(-1).astype(jnp.int32)
    onehot = (flat_e[:, None] == jnp.arange(E, dtype=jnp.int32)[None, :])
    csum = jnp.cumsum(onehot.astype(jnp.int32), axis=0)
    sizes = csum[-1]
    ends = jnp.cumsum(sizes)
    starts = ends - sizes
    rank = jnp.take_along_axis(csum, flat_e[:, None], axis=1)[:, 0] - 1
    inv = (starts[flat_e] + rank).astype(jnp.int32)
    sorted_token = jnp.zeros((P,), jnp.int32).at[inv].set(
        jnp.arange(P, dtype=jnp.int32) // K)

    meta_a = _tile_meta(starts, ends, 0, PC)
    meta_b = _tile_meta(starts, ends, PC, P)
    return sorted_token, inv, meta_a, meta_b


@jax.jit
def kernel(hidden_states, top_k_index, top_k_weights, gate_up_proj, down_proj):
    sorted_token, inv, meta_a, meta_b = _routing_metadata(top_k_index)

    gup_b = gate_up_proj.astype(jnp.bfloat16)
    down_b = down_proj.astype(jnp.bfloat16)

    xs_a = _sc_gather_rows(hidden_states, sorted_token[:PC], PC, H)
    xs_b = _sc_gather_rows(hidden_states, sorted_token[PC:], PC, H)
    ys_a = _grouped_mlp_chunk(xs_a, gup_b, down_b, meta_a, 0)
    ys = _grouped_mlp_chunk(xs_b, gup_b, down_b, meta_b, NBC, prev=ys_a)

    idx2 = inv.reshape(T, K).T
    w_t = top_k_weights.astype(jnp.float32).T
    g_a = _sc_gather_rows(ys, idx2[:, :T2].reshape(-1), P // 2, H)
    g_b = _sc_gather_rows(ys, idx2[:, T2:].reshape(-1), P // 2, H)
    out_a = _combine_half(g_a.reshape(K, T2, H), w_t[:, :T2], 0)
    out = _combine_half(g_b.reshape(K, T2, H), w_t[:, T2:], T2 // BC,
                        prev=out_a)
    return out


# speedup vs baseline: 2.0972x; 1.1386x over previous
import functools

import jax
import jax.numpy as jnp
from jax import lax
from jax.experimental import pallas as pl
from jax.experimental.pallas import tpu as pltpu
from jax.experimental.pallas import tpu_sc as plsc

E = 8
H = 2048
I = 1408
T = 8192
K = 2
P = T * K

BM = 256
NBC = P // BM
TMAXC = NBC + E
BC = 512
GW = 16


def _sc_gather_rows(table, idx, n_rows, row_dim):
    mesh = plsc.VectorSubcoreMesh(core_axis_name="core", subcore_axis_name="subcore")
    info = plsc.get_sparse_core_info()
    nw = info.num_cores * info.num_subcores
    rows_w = n_rows // nw
    nchunks = rows_w // GW

    @functools.partial(
        pl.kernel,
        out_type=jax.ShapeDtypeStruct((n_rows, row_dim), table.dtype),
        mesh=mesh,
        scratch_types=[
            pltpu.VMEM((rows_w,), jnp.int32),
            pltpu.VMEM((2, GW, row_dim), table.dtype),
            pltpu.SemaphoreType.DMA((2,)),
            pltpu.SemaphoreType.DMA((2,)),
        ],
    )
    def gather_kernel(tbl_hbm, i_hbm, o_hbm, idx_v, bufs, gsem, wsem):
        wid = lax.axis_index("subcore") * info.num_cores + lax.axis_index("core")
        base = wid * rows_w
        pltpu.sync_copy(i_hbm.at[pl.ds(base, rows_w)], idx_v)

        def start_gather(c, b):
            pltpu.async_copy(
                tbl_hbm.at[idx_v.at[pl.ds(c * GW, GW)]], bufs.at[b], gsem.at[b])

        def wait_gather(b):
            pltpu.make_async_copy(
                tbl_hbm.at[pl.ds(0, GW)], bufs.at[b], gsem.at[b]).wait()

        def start_write(c, b):
            pltpu.async_copy(
                bufs.at[b], o_hbm.at[pl.ds(base + c * GW, GW)], wsem.at[b])

        def wait_write(b):
            pltpu.make_async_copy(
                bufs.at[b], o_hbm.at[pl.ds(base, GW)], wsem.at[b]).wait()

        start_gather(0, 0)
        start_gather(1, 1)

        @pl.loop(0, nchunks, step=2)
        def _(c):
            for b in range(2):
                wait_gather(b)
                start_write(c + b, b)
            for b in range(2):
                wait_write(b)

                @pl.when(c + 2 + b < nchunks)
                def _():
                    start_gather(c + 2 + b, b)

    return gather_kernel(table, idx)


SCH = 16


def _sc_scatter_pairs(hidden, inv_e, inv_o):
    mesh = plsc.VectorSubcoreMesh(core_axis_name="core", subcore_axis_name="subcore")
    info = plsc.get_sparse_core_info()
    nw = info.num_cores * info.num_subcores
    tok_w = T // nw
    nchunks = tok_w // SCH

    @functools.partial(
        pl.kernel,
        out_type=jax.ShapeDtypeStruct((P, H), jnp.float32),
        mesh=mesh,
        scratch_types=[
            pltpu.VMEM((nchunks, SCH), jnp.int32),
            pltpu.VMEM((nchunks, SCH), jnp.int32),
            pltpu.VMEM((2, SCH, H), jnp.float32),
            pltpu.SemaphoreType.DMA((2,)),
            pltpu.SemaphoreType.DMA((2,)),
        ],
    )
    def scatter_kernel(h_hbm, ie_hbm, io_hbm, o_hbm, ie_v, io_v, bufs, rsem, wsem):
        wid = lax.axis_index("subcore") * info.num_cores + lax.axis_index("core")
        tbase = wid * tok_w
        pltpu.sync_copy(ie_hbm.at[wid], ie_v)
        pltpu.sync_copy(io_hbm.at[wid], io_v)

        def start_read(c, b):
            pltpu.async_copy(
                h_hbm.at[pl.ds(tbase + c * SCH, SCH)], bufs.at[b], rsem.at[b])

        def wait_read(b):
            pltpu.make_async_copy(
                h_hbm.at[pl.ds(0, SCH)], bufs.at[b], rsem.at[b]).wait()

        def start_writes(c, b):
            pltpu.async_copy(bufs.at[b], o_hbm.at[ie_v.at[c]], wsem.at[b])
            pltpu.async_copy(bufs.at[b], o_hbm.at[io_v.at[c]], wsem.at[b])

        def wait_writes(b):
            for _ in range(2):
                pltpu.make_async_copy(
                    bufs.at[b], o_hbm.at[pl.ds(0, SCH)], wsem.at[b]).wait()

        start_read(0, 0)
        start_read(1, 1)

        @pl.loop(0, nchunks, step=2)
        def _(c):
            for b in range(2):
                wait_read(b)
                start_writes(c + b, b)
            for b in range(2):
                wait_writes(b)

                @pl.when(c + 2 + b < nchunks)
                def _():
                    start_read(c + 2 + b, b)

    return scatter_kernel(hidden, inv_e, inv_o)


def _mlp_tile_body(*refs):
    meta_ref, xs_ref, gw_ref, uw_ref, dw_ref = refs[:5]
    o_ref = refs[-1]
    i = pl.program_id(0)
    blk = meta_ref[1, i]
    row_s = meta_ref[2, i]
    row_e = meta_ref[3, i]
    init = meta_ref[4, i]

    @pl.when(row_s < row_e)
    def _():
        x = xs_ref[...].astype(jnp.bfloat16)
        gw = gw_ref[0]
        uw = uw_ref[0]
        dw = dw_ref[0]

        nt = (((1,), (1,)), ((), ()))
        gate = lax.dot_general(x, gw, nt, preferred_element_type=jnp.float32)
        up = lax.dot_general(x, uw, nt, preferred_element_type=jnp.float32)
        h = (gate * jax.nn.sigmoid(gate)) * up
        y = lax.dot_general(h.astype(jnp.bfloat16), dw, nt,
                            preferred_element_type=jnp.float32)

        rows = blk * BM + lax.broadcasted_iota(jnp.int32, (BM, 1), 0)
        mask = (rows >= row_s) & (rows < row_e)
        y = jnp.where(mask, y, 0.0)

        @pl.when(init == 1)
        def _():
            o_ref[...] = y

        @pl.when(init == 0)
        def _():
            o_ref[...] += y


def _grouped_mlp(xs, gup_b, down_b, meta):
    grid_spec = pltpu.PrefetchScalarGridSpec(
        num_scalar_prefetch=1,
        grid=(TMAXC,),
        in_specs=[
            pl.BlockSpec((BM, H), lambda i, m: (m[1, i], 0)),
            pl.BlockSpec((1, I, H), lambda i, m: (m[0, i], 0, 0)),
            pl.BlockSpec((1, I, H), lambda i, m: (m[0, i], 1, 0)),
            pl.BlockSpec((1, H, I), lambda i, m: (m[0, i], 0, 0)),
        ],
        out_specs=pl.BlockSpec((BM, H), lambda i, m: (m[1, i], 0)),
    )
    return pl.pallas_call(
        _mlp_tile_body,
        grid_spec=grid_spec,
        out_shape=jax.ShapeDtypeStruct((P, H), jnp.float32),
    )(meta, xs, gup_b, gup_b, down_b)


def _combine_body(*refs):
    g0_ref, g1_ref, w_ref = refs[:3]
    o_ref = refs[-1]
    w = w_ref[...]
    o_ref[...] = g0_ref[0] * w[0][:, None] + g1_ref[0] * w[1][:, None]


def _combine(g, w_t):
    return pl.pallas_call(
        _combine_body,
        grid=(T // BC,),
        in_specs=[
            pl.BlockSpec((1, BC, H), lambda i: (0, i, 0)),
            pl.BlockSpec((1, BC, H), lambda i: (1, i, 0)),
            pl.BlockSpec((2, BC), lambda i: (0, i)),
        ],
        out_specs=pl.BlockSpec((BC, H), lambda i: (i, 0)),
        out_shape=jax.ShapeDtypeStruct((T, H), jnp.float32),
    )(g, g, w_t)


def _tile_meta(starts, ends, lo, hi):
    s = jnp.clip(starts, lo, hi) - lo
    e = jnp.clip(ends, lo, hi) - lo
    sizes = e - s
    nb = (hi - lo) // BM
    first_blk = s // BM
    last_blk = jnp.maximum(e - 1, 0) // BM
    ntiles = jnp.where(sizes > 0, last_blk - first_blk + 1, 0)
    tile_end = jnp.cumsum(ntiles)
    tile_start = tile_end - ntiles

    ids = jnp.arange(TMAXC, dtype=jnp.int32)
    grp = jnp.sum(ids[:, None] >= tile_end[None, :], axis=1).astype(jnp.int32)
    valid = grp < E
    g = jnp.minimum(grp, E - 1)
    blk = jnp.where(valid, first_blk[g] + (ids - tile_start[g]), nb - 1)
    row_s = jnp.where(valid, jnp.maximum(s[g], blk * BM), 0)
    row_e = jnp.where(valid, jnp.minimum(e[g], blk * BM + BM), 0)
    init = jnp.concatenate([jnp.ones((1,), jnp.int32),
                            (blk[1:] != blk[:-1]).astype(jnp.int32)])
    return jnp.stack([g.astype(jnp.int32), blk.astype(jnp.int32),
                      row_s.astype(jnp.int32), row_e.astype(jnp.int32), init])


def _routing_metadata(top_k_index):
    flat_e = top_k_index.reshape(-1).astype(jnp.int32)
    onehot = (flat_e[:, None] == jnp.arange(E, dtype=jnp.int32)[None, :])
    csum = jnp.cumsum(onehot.astype(jnp.int32), axis=0)
    sizes = csum[-1]
    ends = jnp.cumsum(sizes)
    starts = ends - sizes
    rank = jnp.take_along_axis(csum, flat_e[:, None], axis=1)[:, 0] - 1
    inv = (starts[flat_e] + rank).astype(jnp.int32)

    meta = _tile_meta(starts, ends, 0, P)
    return inv, meta


@jax.jit
def kernel(hidden_states, top_k_index, top_k_weights, gate_up_proj, down_proj):
    inv, meta = _routing_metadata(top_k_index)

    gup_b = gate_up_proj.astype(jnp.bfloat16)
    down_b = down_proj.astype(jnp.bfloat16)

    nw = 32
    inv2 = inv.reshape(T, K)
    inv_e = inv2[:, 0].reshape(nw, T // (nw * SCH), SCH)
    inv_o = inv2[:, 1].reshape(nw, T // (nw * SCH), SCH)
    xs = _sc_scatter_pairs(hidden_states, inv_e, inv_o)
    ys = _grouped_mlp(xs, gup_b, down_b, meta)
    idx2 = inv.reshape(T, K).T.reshape(-1)
    g = _sc_gather_rows(ys, idx2, P, H).reshape(K, T, H)
    w_t = top_k_weights.astype(jnp.float32).T
    return _combine(g, w_t)
